# baseline (speedup 1.0000x reference)
import jax
import jax.numpy as jnp
import numpy as np
from jax import lax
from jax.experimental import pallas as pl
from jax.experimental.pallas import tpu as pltpu
from jax.experimental.pallas import tpu_sc as plsc

VOCAB = 100000
VOCAB_PAD = 100096
NTILECOL = VOCAB_PAD // 128
EMBED = 64
BATCH = 4096
SEQ = 50
NBT = BATCH // 128
NW = 32

FLAT_TABLE = VOCAB_PAD * EMBED
FLAT_IDX = SEQ * BATCH

_i16 = None


def _iota16():
    return lax.iota(jnp.int32, 16)


def _splat(x):
    return jnp.zeros((16,), jnp.int32) + x


def _detile_body(idx_hbm, table_hbm, tflat_hbm, iflat_hbm,
                 inbuf, outbuf, ibuf, sem_in, sem_out, sem_ib, sem_iw):
    w = lax.axis_index("s") * 2 + lax.axis_index("c")

    ilds = []
    for st in range(7):
        rows = 8 if st < 6 else 2
        ilds.append(pltpu.async_copy(
            idx_hbm.at[pl.ds(st * 8, rows),
                       pl.ds(pl.multiple_of(w * 128, 128), 128)],
            ibuf.at[pl.ds(st * 8, rows)], sem_ib))
    for c in ilds:
        c.wait()
    iwrites = []
    for s in range(SEQ):
        iwrites.append(pltpu.async_copy(
            ibuf.at[s],
            iflat_hbm.at[pl.ds((s * NBT + w) * 128, 128)], sem_iw))

    iota = _iota16()

    def transpose_col(p):
        def bstep(b, _):
            boff = b * EMBED
            for k in range(4):
                dvec = iota + (k * 16)
                vals = plsc.load_gather(inbuf.at[p], [dvec, _splat(b)])
                outbuf[pl.ds(boff + k * 16, 16)] = vals
            return 0

        def bstep_o(b, _):
            boff = 8192 + b * EMBED
            for k in range(4):
                dvec = iota + (k * 16)
                vals = plsc.load_gather(inbuf.at[p], [dvec, _splat(b)])
                outbuf[pl.ds(boff + k * 16, 16)] = vals
            return 0

        return bstep if p == 0 else bstep_o

    def col_of(i):
        return i * NW + w

    def in_copy(i, p):
        c = col_of(i)
        return pltpu.async_copy(
            table_hbm.at[:, pl.ds(pl.multiple_of(c * 128, 128), 128)],
            inbuf.at[p], sem_in)

    def out_copy(i, p):
        c = col_of(i)
        return pltpu.async_copy(
            outbuf.at[pl.ds(p * 8192, 8192)],
            tflat_hbm.at[pl.ds(pl.multiple_of(c * 8192, 8), 8192)], sem_out)

    NCOL = 24
    loads = {0: in_copy(0, 0)}
    stores = {}
    for i in range(NCOL):
        p = i % 2
        loads[i].wait()
        if i + 1 < NCOL:
            loads[i + 1] = in_copy(i + 1, (i + 1) % 2)
        if i >= 2:
            stores[i - 2].wait()
        lax.fori_loop(0, 128, transpose_col(p), 0)
        stores[i] = out_copy(i, p)
    stores[NCOL - 2].wait()
    stores[NCOL - 1].wait()

    @pl.when(w < NTILECOL - NCOL * NW)
    def _tail():
        c = NCOL * NW + w
        pltpu.async_copy(
            table_hbm.at[:, pl.ds(pl.multiple_of(c * 128, 128), 128)],
            inbuf.at[0], sem_in).wait()
        lax.fori_loop(0, 128, transpose_col(0), 0)
        pltpu.async_copy(
            outbuf.at[pl.ds(0, 8192)],
            tflat_hbm.at[pl.ds(pl.multiple_of(c * 8192, 8), 8192)],
            sem_out).wait()

    for c in iwrites:
        c.wait()


def _gather_body(iflat_hbm, tflat_hbm, out_hbm,
                 idxbuf, rowsbuf, tbuf, sem_i, sem_g, sem_s):
    w = lax.axis_index("s") * 2 + lax.axis_index("c")
    iota = _iota16()

    def idx_load(s):
        return pltpu.async_copy(
            iflat_hbm.at[pl.ds((s * NBT + w) * 128, 128)],
            idxbuf.at[s % 2], sem_i)

    def gather(s):
        return pltpu.async_copy(
            tflat_hbm.at[idxbuf.at[s % 2]], rowsbuf.at[s % 2], sem_g)

    idx_loads = {0: idx_load(0), 1: idx_load(1)}
    idx_loads[0].wait()
    gathers = {0: gather(0)}
    stores = {}
    for s in range(SEQ):
        p = s % 2
        gathers[s].wait()
        if s + 1 < SEQ:
            idx_loads[s + 1].wait()
            gathers[s + 1] = gather(s + 1)
        if s + 2 < SEQ:
            idx_loads[s + 2] = idx_load(s + 2)
        if s >= 2:
            for c in stores[s - 2]:
                c.wait()

        def dstep(d, _):
            dt = lax.shift_right_logical(d, 3)
            d8 = lax.bitwise_and(d, 7)
            for k in range(8):
                bvec = iota + (k * 16)
                vals = plsc.load_gather(rowsbuf.at[p], [bvec, _splat(d)])
                tbuf[p, dt, d8, pl.ds(k * 16, 16)] = vals
            return 0

        lax.fori_loop(0, EMBED, dstep, 0)
        ss = []
        for dt in range(8):
            ss.append(pltpu.async_copy(
                tbuf.at[p, dt], out_hbm.at[s, dt, w], sem_s))
        stores[s] = ss
    for ss in (stores[SEQ - 2], stores[SEQ - 1]):
        for c in ss:
            c.wait()


@jax.jit
def _lookup(idx_t, table_t):
    mesh = plsc.VectorSubcoreMesh(core_axis_name="c", subcore_axis_name="s")
    tflat, iflat = pl.kernel(
        _detile_body,
        out_type=(
            jax.ShapeDtypeStruct((FLAT_TABLE,), jnp.float32),
            jax.ShapeDtypeStruct((FLAT_IDX,), jnp.int32),
        ),
        mesh=mesh,
        scratch_types=[
            pltpu.VMEM((2, EMBED, 128), jnp.float32),
            pltpu.VMEM((16384,), jnp.float32),
            pltpu.VMEM((56, 128), jnp.int32),
            pltpu.SemaphoreType.DMA,
            pltpu.SemaphoreType.DMA,
            pltpu.SemaphoreType.DMA,
            pltpu.SemaphoreType.DMA,
        ],
        compiler_params=pltpu.CompilerParams(use_tc_tiling_on_sc=True, needs_layout_passes=False),
    )(idx_t, table_t)

    table_lin = jnp.reshape(tflat, (VOCAB_PAD, EMBED))
    mesh2 = plsc.VectorSubcoreMesh(core_axis_name="c", subcore_axis_name="s")
    out5 = pl.kernel(
        _gather_body,
        out_type=jax.ShapeDtypeStruct((SEQ, 8, NBT, 8, 128), jnp.float32),
        mesh=mesh2,
        scratch_types=[
            pltpu.VMEM((2, 128), jnp.int32),
            pltpu.VMEM((2, 128, EMBED), jnp.float32),
            pltpu.VMEM((2, 8, 8, 128), jnp.float32),
            pltpu.SemaphoreType.DMA,
            pltpu.SemaphoreType.DMA,
            pltpu.SemaphoreType.DMA,
        ],
        compiler_params=pltpu.CompilerParams(use_tc_tiling_on_sc=False, needs_layout_passes=False),
    )(iflat, table_lin)
    return out5


def kernel(token_type_ids, table):
    idx_t = jnp.transpose(token_type_ids, (1, 0))
    table_t = jnp.transpose(table, (1, 0))
    out5 = _lookup(idx_t, table_t)
    out = jnp.transpose(out5, (2, 4, 0, 1, 3))
    return jnp.reshape(out, (BATCH, SEQ, EMBED))


# speedup vs baseline: 3.0561x; 3.0561x over previous
import jax
import jax.numpy as jnp
from jax import lax
from jax.experimental import pallas as pl
from jax.experimental.pallas import tpu as pltpu
from jax.experimental.pallas import tpu_sc as plsc

VOCAB = 100000
PLANE_STRIDE = 100096
EMBED = 64
BATCH = 4096
SEQ = 50
NBT = BATCH // 128
NW = 32
NTILECOL = PLANE_STRIDE // 128

FLAT_TABLE = EMBED * PLANE_STRIDE
FLAT_IDX = SEQ * BATCH


def _detile_body(idx_hbm, table_hbm, tpl_hbm, iflat_hbm,
                 inbuf, ibuf, sem_in, sem_pw, sem_ib, sem_iw):
    w = lax.axis_index("s") * 2 + lax.axis_index("c")

    ilds = []
    for st in range(7):
        rows = 8 if st < 6 else 2
        ilds.append(pltpu.async_copy(
            idx_hbm.at[pl.ds(st * 8, rows),
                       pl.ds(pl.multiple_of(w * 128, 128), 128)],
            ibuf.at[pl.ds(st * 8, rows)], sem_ib))
    for c in ilds:
        c.wait()
    iwrites = []
    for s in range(SEQ):
        iwrites.append(pltpu.async_copy(
            ibuf.at[s],
            iflat_hbm.at[pl.ds((s * NBT + w) * 128, 128)], sem_iw))

    def plane_fire(c, p):
        def fire(d, _):
            pltpu.async_copy(
                inbuf.at[p, d],
                tpl_hbm.at[pl.ds(d * PLANE_STRIDE + c * 128, 128)], sem_pw)
            return 0
        lax.fori_loop(0, EMBED, fire, 0)

    def plane_drain(c, p):
        def drain(d, _):
            pltpu.make_async_copy(
                inbuf.at[p, d],
                tpl_hbm.at[pl.ds(d * PLANE_STRIDE + c * 128, 128)],
                sem_pw).wait()
            return 0
        lax.fori_loop(0, EMBED, drain, 0)

    def in_copy(i, p):
        c = i * NW + w
        return pltpu.async_copy(
            table_hbm.at[:, pl.ds(pl.multiple_of(c * 128, 128), 128)],
            inbuf.at[p], sem_in)

    NCOL = 24
    loads = {0: in_copy(0, 0)}
    for i in range(NCOL):
        p = i % 2
        loads[i].wait()
        if i >= 1:
            plane_drain((i - 1) * NW + w, (i - 1) % 2)
        if i + 1 < NCOL:
            loads[i + 1] = in_copy(i + 1, (i + 1) % 2)
        plane_fire(i * NW + w, p)
    plane_drain((NCOL - 1) * NW + w, (NCOL - 1) % 2)

    @pl.when(w < NTILECOL - NCOL * NW)
    def _tail():
        c = NCOL * NW + w
        pltpu.async_copy(
            table_hbm.at[:, pl.ds(pl.multiple_of(c * 128, 128), 128)],
            inbuf.at[0], sem_in).wait()
        plane_fire(c, 0)
        plane_drain(c, 0)

    for c in iwrites:
        c.wait()


def _gather_body(iflat_hbm, tpl_hbm, out_hbm,
                 idxrow, planebuf, sbuf, sem_p, sem_i, sem_s):
    w = lax.axis_index("s") * 2 + lax.axis_index("c")

    def do_plane(d):
        dt = lax.shift_right_logical(d, 3)
        d8 = lax.bitwise_and(d, 7)
        pltpu.async_copy(
            tpl_hbm.at[pl.ds(pl.multiple_of(d * PLANE_STRIDE, 8), VOCAB)],
            planebuf, sem_p).wait()

        def idx_load(s):
            return pltpu.async_copy(
                iflat_hbm.at[pl.ds(s * BATCH, BATCH)], idxrow.at[s % 2],
                sem_i)

        iloads = {0: idx_load(0), 1: idx_load(1)}
        stores = {}
        for s in range(SEQ):
            p = s % 2
            iloads[s].wait()
            if s + 2 < SEQ:
                iloads[s + 2] = idx_load(s + 2)
            if s >= 2:
                stores[s - 2].wait()

            @plsc.parallel_loop(0, BATCH, step=16, unroll=8)
            def _gather(j):
                ivals = idxrow[p, pl.ds(j, 16)]
                vals = plsc.load_gather(planebuf, [ivals])
                bt = lax.shift_right_logical(j, 7)
                off = lax.bitwise_and(j, 127)
                sbuf[p, bt, pl.ds(off, 16)] = vals

            stores[s] = pltpu.async_copy(
                sbuf.at[p], out_hbm.at[s, dt, :, d8, :], sem_s)
        stores[SEQ - 2].wait()
        stores[SEQ - 1].wait()

    do_plane(w)
    do_plane(w + 32)


@jax.jit
def _lookup(idx_t, table_t):
    mesh = plsc.VectorSubcoreMesh(core_axis_name="c", subcore_axis_name="s")
    tpl, iflat = pl.kernel(
        _detile_body,
        out_type=(
            jax.ShapeDtypeStruct((FLAT_TABLE,), jnp.float32),
            jax.ShapeDtypeStruct((FLAT_IDX,), jnp.int32),
        ),
        mesh=mesh,
        scratch_types=[
            pltpu.VMEM((2, EMBED, 128), jnp.float32),
            pltpu.VMEM((56, 128), jnp.int32),
            pltpu.SemaphoreType.DMA,
            pltpu.SemaphoreType.DMA,
            pltpu.SemaphoreType.DMA,
            pltpu.SemaphoreType.DMA,
        ],
        compiler_params=pltpu.CompilerParams(
            use_tc_tiling_on_sc=True, needs_layout_passes=False),
    )(idx_t, table_t)

    mesh2 = plsc.VectorSubcoreMesh(core_axis_name="c", subcore_axis_name="s")
    out5 = pl.kernel(
        _gather_body,
        out_type=jax.ShapeDtypeStruct((SEQ, 8, NBT, 8, 128), jnp.float32),
        mesh=mesh2,
        scratch_types=[
            pltpu.VMEM((2, BATCH), jnp.int32),
            pltpu.VMEM((VOCAB,), jnp.float32),
            pltpu.VMEM((2, NBT, 128), jnp.float32),
            pltpu.SemaphoreType.DMA,
            pltpu.SemaphoreType.DMA,
            pltpu.SemaphoreType.DMA,
        ],
        compiler_params=pltpu.CompilerParams(
            use_tc_tiling_on_sc=False, needs_layout_passes=False),
    )(iflat, tpl)
    return out5


def kernel(token_type_ids, table):
    idx_t = jnp.transpose(token_type_ids, (1, 0))
    table_t = jnp.transpose(table, (1, 0))
    out5 = _lookup(idx_t, table_t)
    out = jnp.transpose(out5, (2, 4, 0, 1, 3))
    return jnp.reshape(out, (BATCH, SEQ, EMBED))
